# Initial kernel scaffold; baseline (speedup 1.0000x reference)
import jax
import jax.numpy as jnp
from jax.experimental import pallas as pl


def kernel(x, edge_index, W1, b1, W2, b2, W3, b3):
    raise NotImplementedError("write your pallas kernel here")


# speedup vs baseline: 9.0840x; 9.0840x over previous
import functools

import jax
import jax.numpy as jnp
from jax import lax
from jax.experimental import pallas as pl
from jax.experimental.pallas import tpu as pltpu
from jax.experimental.pallas import tpu_sc as plsc

N = 10000
E = 320000
D = 128
NC, NS = 2, 16
NW = NC * NS
CHUNK = 128
NCH = 80
EPAD = NW * NCH * CHUNK
NPAD = 10240
RPT = NPAD // NS

_MESH = plsc.VectorSubcoreMesh(core_axis_name="c", subcore_axis_name="s")


@functools.partial(
    pl.kernel,
    out_type=jax.ShapeDtypeStruct((NC, NPAD), jnp.float32),
    mesh=_MESH,
    scratch_types=[
        pltpu.VMEM((NCH, CHUNK), jnp.int32),
        pltpu.VMEM((CHUNK,), jnp.float32),
        pltpu.VMEM((RPT,), jnp.float32),
        pltpu.VMEM_SHARED((NPAD,), jnp.float32),
    ],
)
def _deg_kernel(dst_hbm, deg_hbm, dst_v, ones_v, buf_v, deg_sh):
    c = lax.axis_index("c")
    s = lax.axis_index("s")
    w = c * NS + s

    pltpu.sync_copy(dst_hbm.at[pl.ds(w * NCH, NCH)], dst_v)

    def fill_ones(i, _):
        ones_v[pl.ds(i * 16, 16)] = jnp.full((16,), 1.0, jnp.float32)
        return 0

    lax.fori_loop(0, CHUNK // 16, fill_ones, 0)

    def fill_zero(i, _):
        buf_v[pl.ds(i * 16, 16)] = jnp.zeros((16,), jnp.float32)
        return 0

    lax.fori_loop(0, RPT // 16, fill_zero, 0)

    pltpu.sync_copy(buf_v, deg_sh.at[pl.ds(s * RPT, RPT)])
    plsc.subcore_barrier()

    def body(j, _):
        pltpu.sync_copy(ones_v, deg_sh.at[dst_v.at[j]], add=True)
        return 0

    lax.fori_loop(0, NCH, body, 0)
    plsc.subcore_barrier()

    pltpu.sync_copy(deg_sh.at[pl.ds(s * RPT, RPT)], buf_v)
    pltpu.sync_copy(buf_v, deg_hbm.at[c, pl.ds(s * RPT, RPT)])


@functools.partial(
    pl.kernel,
    out_type=jax.ShapeDtypeStruct((NC, NPAD, D), jnp.float32),
    mesh=_MESH,
    scratch_types=[
        pltpu.VMEM((NCH, CHUNK), jnp.int32),
        pltpu.VMEM((NCH, CHUNK), jnp.int32),
        pltpu.VMEM((CHUNK, D), jnp.float32),
        pltpu.VMEM_SHARED((NPAD, D), jnp.float32),
        pltpu.SemaphoreType.DMA,
    ],
)
def _agg_kernel(y_hbm, src_hbm, dst_hbm, zeros_hbm, agg_hbm,
                src_v, dst_v, rows_v, agg_sh, sem):
    c = lax.axis_index("c")
    s = lax.axis_index("s")
    w = c * NS + s

    pltpu.sync_copy(src_hbm.at[pl.ds(w * NCH, NCH)], src_v)
    pltpu.sync_copy(dst_hbm.at[pl.ds(w * NCH, NCH)], dst_v)

    pltpu.sync_copy(zeros_hbm, rows_v)
    for t in range(RPT // CHUNK):
        pltpu.sync_copy(rows_v, agg_sh.at[pl.ds(s * RPT + t * CHUNK, CHUNK)])
    plsc.subcore_barrier()

    def body(j, _):
        pltpu.async_copy(y_hbm.at[src_v.at[j]], rows_v, sem).wait()
        pltpu.sync_copy(rows_v, agg_sh.at[dst_v.at[j]], add=True)
        return 0

    lax.fori_loop(0, NCH, body, 0)
    plsc.subcore_barrier()

    for t in range(RPT // CHUNK):
        base = s * RPT + t * CHUNK
        pltpu.sync_copy(agg_sh.at[pl.ds(base, CHUNK)], rows_v)
        pltpu.sync_copy(rows_v, agg_hbm.at[c, pl.ds(base, CHUNK)])


_BLK = 1000


def _tc1_body(deg_ref, x_ref, w_ref, y_ref, dinv_ref):
    d = deg_ref[0] + deg_ref[1] + 1.0
    dinv = lax.rsqrt(d)
    xw = jnp.dot(x_ref[...], w_ref[...], preferred_element_type=jnp.float32)
    y_ref[...] = dinv * xw
    dinv_ref[...] = dinv


def _tc_mid_body(agg_ref, y_ref, dinv_ref, b_ref, w_ref, y2_ref):
    a = agg_ref[0] + agg_ref[1] + y_ref[...]
    h = jnp.maximum(dinv_ref[...] * a + b_ref[...], 0.0)
    y2_ref[...] = dinv_ref[...] * jnp.dot(
        h, w_ref[...], preferred_element_type=jnp.float32)


def _tc_out_body(agg_ref, y_ref, dinv_ref, b_ref, w_ref, b3_ref, out_ref):
    a = agg_ref[0] + agg_ref[1] + y_ref[...]
    h = jnp.maximum(dinv_ref[...] * a + b_ref[...], 0.0)
    out_ref[...] = jnp.dot(
        h, w_ref[...], preferred_element_type=jnp.float32) + b3_ref[...]


def _tc1(deg, x, w1):
    return pl.pallas_call(
        _tc1_body,
        grid=(N // _BLK,),
        in_specs=[
            pl.BlockSpec((NC, _BLK, 1), lambda j: (0, j, 0)),
            pl.BlockSpec((_BLK, D), lambda j: (j, 0)),
            pl.BlockSpec((D, D), lambda j: (0, 0)),
        ],
        out_specs=[
            pl.BlockSpec((_BLK, D), lambda j: (j, 0)),
            pl.BlockSpec((_BLK, 1), lambda j: (j, 0)),
        ],
        out_shape=[
            jax.ShapeDtypeStruct((N, D), jnp.float32),
            jax.ShapeDtypeStruct((N, 1), jnp.float32),
        ],
    )(deg, x, w1)


def _tc_mid(agg, y, dinv, b1, w2):
    return pl.pallas_call(
        _tc_mid_body,
        grid=(N // _BLK,),
        in_specs=[
            pl.BlockSpec((NC, _BLK, D), lambda j: (0, j, 0)),
            pl.BlockSpec((_BLK, D), lambda j: (j, 0)),
            pl.BlockSpec((_BLK, 1), lambda j: (j, 0)),
            pl.BlockSpec((1, D), lambda j: (0, 0)),
            pl.BlockSpec((D, D), lambda j: (0, 0)),
        ],
        out_specs=pl.BlockSpec((_BLK, D), lambda j: (j, 0)),
        out_shape=jax.ShapeDtypeStruct((N, D), jnp.float32),
    )(agg, y, dinv, b1, w2)


def _tc_out(agg, y, dinv, b2, w3, b3):
    return pl.pallas_call(
        _tc_out_body,
        grid=(N // _BLK,),
        in_specs=[
            pl.BlockSpec((NC, _BLK, D), lambda j: (0, j, 0)),
            pl.BlockSpec((_BLK, D), lambda j: (j, 0)),
            pl.BlockSpec((_BLK, 1), lambda j: (j, 0)),
            pl.BlockSpec((1, D), lambda j: (0, 0)),
            pl.BlockSpec((D, w3.shape[1]), lambda j: (0, 0)),
            pl.BlockSpec((1, w3.shape[1]), lambda j: (0, 0)),
        ],
        out_specs=pl.BlockSpec((_BLK, w3.shape[1]), lambda j: (j, 0)),
        out_shape=jax.ShapeDtypeStruct((N, w3.shape[1]), jnp.float32),
    )(agg, y, dinv, b2, w3, b3)


@jax.jit
def kernel(x, edge_index, W1, b1, W2, b2, W3, b3):
    pad = EPAD - E
    src2d = jnp.concatenate(
        [edge_index[0], jnp.zeros((pad,), jnp.int32)]).reshape(EPAD // CHUNK, CHUNK)
    dst2d = jnp.concatenate(
        [edge_index[1], jnp.full((pad,), N, jnp.int32)]).reshape(EPAD // CHUNK, CHUNK)
    zeros128 = jnp.zeros((CHUNK, D), jnp.float32)

    deg = _deg_kernel(dst2d).reshape(NC, NPAD, 1)

    y1, dinv = _tc1(deg, x, W1)
    agg1 = _agg_kernel(y1, src2d, dst2d, zeros128)
    y2 = _tc_mid(agg1, y1, dinv, b1.reshape(1, D), W2)
    agg2 = _agg_kernel(y2, src2d, dst2d, zeros128)
    return _tc_out(agg2, y2, dinv, b2.reshape(1, D), W3,
                   b3.reshape(1, W3.shape[1]))


# speedup vs baseline: 10.5964x; 1.1665x over previous
import functools

import jax
import jax.numpy as jnp
from jax import lax
from jax.experimental import pallas as pl
from jax.experimental.pallas import tpu as pltpu
from jax.experimental.pallas import tpu_sc as plsc

N = 10000
E = 320000
D = 128
DH = D // 2
NC, NS = 2, 16
NW = NC * NS
CHUNK = 128
NCHT = 80
EPAD = NW * NCHT * CHUNK
NCHS = EPAD // CHUNK // NS
NPAD = 10240
RPT = NPAD // NS

_MESH = plsc.VectorSubcoreMesh(core_axis_name="c", subcore_axis_name="s")


@functools.partial(
    pl.kernel,
    out_type=jax.ShapeDtypeStruct((NC, NPAD), jnp.float32),
    mesh=_MESH,
    scratch_types=[
        pltpu.VMEM((NCHT, CHUNK), jnp.int32),
        pltpu.VMEM((CHUNK,), jnp.float32),
        pltpu.VMEM((RPT,), jnp.float32),
        pltpu.VMEM_SHARED((NPAD,), jnp.float32),
    ],
)
def _deg_kernel(dst_hbm, deg_hbm, dst_v, ones_v, buf_v, deg_sh):
    c = lax.axis_index("c")
    s = lax.axis_index("s")
    w = c * NS + s

    pltpu.sync_copy(dst_hbm.at[pl.ds(w * NCHT, NCHT)], dst_v)

    def fill_ones(i, _):
        ones_v[pl.ds(i * 16, 16)] = jnp.full((16,), 1.0, jnp.float32)
        return 0

    lax.fori_loop(0, CHUNK // 16, fill_ones, 0)

    def fill_zero(i, _):
        buf_v[pl.ds(i * 16, 16)] = jnp.zeros((16,), jnp.float32)
        return 0

    lax.fori_loop(0, RPT // 16, fill_zero, 0)

    pltpu.sync_copy(buf_v, deg_sh.at[pl.ds(s * RPT, RPT)])
    plsc.subcore_barrier()

    def body(j, _):
        pltpu.sync_copy(ones_v, deg_sh.at[dst_v.at[j]], add=True)
        return 0

    lax.fori_loop(0, NCHT, body, 0)
    plsc.subcore_barrier()

    pltpu.sync_copy(deg_sh.at[pl.ds(s * RPT, RPT)], buf_v)
    pltpu.sync_copy(buf_v, deg_hbm.at[c, pl.ds(s * RPT, RPT)])


@functools.partial(
    pl.kernel,
    out_type=jax.ShapeDtypeStruct((NC, NPAD, DH), jnp.float32),
    mesh=_MESH,
    scratch_types=[
        pltpu.VMEM((2, NCHS, CHUNK), jnp.int32),
        pltpu.VMEM((CHUNK, DH), jnp.float32),
        pltpu.VMEM((CHUNK, DH), jnp.float32),
        pltpu.VMEM_SHARED((NPAD, DH), jnp.float32),
        pltpu.SemaphoreType.DMA,
        pltpu.SemaphoreType.DMA,
    ],
    compiler_params=pltpu.CompilerParams(use_tc_tiling_on_sc=False),
)
def _agg_kernel(y2_hbm, srch_hbm, dst_hbm, zeros_hbm, agg_hbm,
                idx_v, rows_a, rows_b, agg_sh, sem_a, sem_b):
    c = lax.axis_index("c")
    s = lax.axis_index("s")

    pltpu.sync_copy(srch_hbm.at[c, pl.ds(s * NCHS, NCHS)], idx_v.at[0])
    pltpu.sync_copy(dst_hbm.at[pl.ds(s * NCHS, NCHS)], idx_v.at[1])
    src_v = idx_v.at[0]
    dst_v = idx_v.at[1]

    pltpu.sync_copy(zeros_hbm, rows_a)

    def zinit(t, _):
        pltpu.sync_copy(rows_a, agg_sh.at[pl.ds(s * RPT + t * CHUNK, CHUNK)])
        return 0

    lax.fori_loop(0, RPT // CHUNK, zinit, 0)
    plsc.subcore_barrier()

    def body(i, _):
        j0 = 2 * i
        j1 = j0 + 1
        ca = pltpu.async_copy(y2_hbm.at[src_v.at[j0]], rows_a, sem_a)
        cb = pltpu.async_copy(y2_hbm.at[src_v.at[j1]], rows_b, sem_b)
        ca.wait()
        pltpu.sync_copy(rows_a, agg_sh.at[dst_v.at[j0]], add=True)
        cb.wait()
        pltpu.sync_copy(rows_b, agg_sh.at[dst_v.at[j1]], add=True)
        return 0

    lax.fori_loop(0, NCHS // 2, body, 0)
    plsc.subcore_barrier()

    def wout(t, _):
        base = s * RPT + t * CHUNK
        pltpu.sync_copy(agg_sh.at[pl.ds(base, CHUNK)], rows_a)
        pltpu.sync_copy(rows_a, agg_hbm.at[c, pl.ds(base, CHUNK)])
        return 0

    lax.fori_loop(0, RPT // CHUNK, wout, 0)


_BLK = 1000


def _tc1_body(deg_ref, x_ref, w_ref, y_ref, dinv_ref):
    d = deg_ref[0] + deg_ref[1] + 1.0
    dinv = lax.rsqrt(d)
    xw = jnp.dot(x_ref[...], w_ref[...], preferred_element_type=jnp.float32)
    y_ref[...] = dinv * xw
    dinv_ref[...] = dinv


def _tc_mid_body(agg_ref, y_ref, dinv_ref, b_ref, w_ref, y2_ref):
    a = jnp.concatenate([agg_ref[0], agg_ref[1]], axis=-1) + y_ref[...]
    h = jnp.maximum(dinv_ref[...] * a + b_ref[...], 0.0)
    y2_ref[...] = dinv_ref[...] * jnp.dot(
        h, w_ref[...], preferred_element_type=jnp.float32)


def _tc_out_body(agg_ref, y_ref, dinv_ref, b_ref, w_ref, b3_ref, out_ref):
    a = jnp.concatenate([agg_ref[0], agg_ref[1]], axis=-1) + y_ref[...]
    h = jnp.maximum(dinv_ref[...] * a + b_ref[...], 0.0)
    out_ref[...] = jnp.dot(
        h, w_ref[...], preferred_element_type=jnp.float32) + b3_ref[...]


def _tc1(deg, x, w1):
    return pl.pallas_call(
        _tc1_body,
        grid=(N // _BLK,),
        in_specs=[
            pl.BlockSpec((NC, _BLK, 1), lambda j: (0, j, 0)),
            pl.BlockSpec((_BLK, D), lambda j: (j, 0)),
            pl.BlockSpec((D, D), lambda j: (0, 0)),
        ],
        out_specs=[
            pl.BlockSpec((_BLK, D), lambda j: (j, 0)),
            pl.BlockSpec((_BLK, 1), lambda j: (j, 0)),
        ],
        out_shape=[
            jax.ShapeDtypeStruct((N, D), jnp.float32),
            jax.ShapeDtypeStruct((N, 1), jnp.float32),
        ],
    )(deg, x, w1)


def _tc_mid(agg, y, dinv, b1, w2):
    return pl.pallas_call(
        _tc_mid_body,
        grid=(N // _BLK,),
        in_specs=[
            pl.BlockSpec((NC, _BLK, DH), lambda j: (0, j, 0)),
            pl.BlockSpec((_BLK, D), lambda j: (j, 0)),
            pl.BlockSpec((_BLK, 1), lambda j: (j, 0)),
            pl.BlockSpec((1, D), lambda j: (0, 0)),
            pl.BlockSpec((D, D), lambda j: (0, 0)),
        ],
        out_specs=pl.BlockSpec((_BLK, D), lambda j: (j, 0)),
        out_shape=jax.ShapeDtypeStruct((N, D), jnp.float32),
    )(agg, y, dinv, b1, w2)


def _tc_out(agg, y, dinv, b2, w3, b3):
    return pl.pallas_call(
        _tc_out_body,
        grid=(N // _BLK,),
        in_specs=[
            pl.BlockSpec((NC, _BLK, DH), lambda j: (0, j, 0)),
            pl.BlockSpec((_BLK, D), lambda j: (j, 0)),
            pl.BlockSpec((_BLK, 1), lambda j: (j, 0)),
            pl.BlockSpec((1, D), lambda j: (0, 0)),
            pl.BlockSpec((D, w3.shape[1]), lambda j: (0, 0)),
            pl.BlockSpec((1, w3.shape[1]), lambda j: (0, 0)),
        ],
        out_specs=pl.BlockSpec((_BLK, w3.shape[1]), lambda j: (j, 0)),
        out_shape=jax.ShapeDtypeStruct((N, w3.shape[1]), jnp.float32),
    )(agg, y, dinv, b2, w3, b3)


@jax.jit
def kernel(x, edge_index, W1, b1, W2, b2, W3, b3):
    pad = EPAD - E
    src_p = jnp.concatenate([edge_index[0], jnp.zeros((pad,), jnp.int32)])
    dst_p = jnp.concatenate([edge_index[1], jnp.full((pad,), N, jnp.int32)])
    srch2d = jnp.stack([2 * src_p, 2 * src_p + 1]).reshape(
        2, EPAD // CHUNK, CHUNK)
    dst2d = dst_p.reshape(EPAD // CHUNK, CHUNK)
    zerosh = jnp.zeros((CHUNK, DH), jnp.float32)

    deg = _deg_kernel(dst2d).reshape(NC, NPAD, 1)

    y1, dinv = _tc1(deg, x, W1)
    agg1 = _agg_kernel(y1.reshape(2 * N, DH), srch2d, dst2d, zerosh)
    y2 = _tc_mid(agg1, y1, dinv, b1.reshape(1, D), W2)
    agg2 = _agg_kernel(y2.reshape(2 * N, DH), srch2d, dst2d, zerosh)
    return _tc_out(agg2, y2, dinv, b2.reshape(1, D), W3,
                   b3.reshape(1, W3.shape[1]))
